# Initial kernel scaffold; baseline (speedup 1.0000x reference)
import jax
import jax.numpy as jnp
from jax.experimental import pallas as pl


def kernel(x, edge_index):
    raise NotImplementedError("write your pallas kernel here")


# speedup vs baseline: 10.1592x; 10.1592x over previous
import functools

import jax
import jax.numpy as jnp
from jax import lax
from jax.experimental import pallas as pl
from jax.experimental.pallas import tpu as pltpu
from jax.experimental.pallas import tpu_sc as plsc

N = 10000
D = 128
E = 320000
NC, NS = 2, 16
NT = NC * NS
CH = 128
CPT = 80
EPAD = NT * CPT * CH
NSP = 10112
RPT = NSP // NS
ZR = RPT // 8
SB = 14
SM = (1 << SB) - 1

_mesh = plsc.VectorSubcoreMesh(
    core_axis_name="c", subcore_axis_name="s", num_cores=NC, num_subcores=NS)


@functools.partial(
    pl.kernel,
    out_type=jax.ShapeDtypeStruct((NC, NSP, 16), jnp.float32),
    mesh=_mesh,
    scratch_types=[
        pltpu.VMEM((CPT, CH), jnp.int32),
        pltpu.VMEM((CH,), jnp.int32),
        pltpu.VMEM((CH, 16), jnp.float32),
        pltpu.VMEM((ZR, 16), jnp.float32),
        pltpu.VMEM_SHARED((NSP, 16), jnp.float32),
    ],
)
def _sc_degree(pk_hbm, degw_hbm, pk_v, idx_v, ones_v, zb_v, degsp):
    cid = lax.axis_index("c")
    sid = lax.axis_index("s")
    w = cid * NS + sid
    e0 = jnp.where(lax.iota(jnp.int32, 16) == 0,
                   jnp.float32(1), jnp.float32(0))
    z16 = jnp.zeros((16,), jnp.float32)

    def fill_ones(j, carry):
        ones_v[j, :] = e0
        return carry
    lax.fori_loop(0, CH, fill_ones, 0)

    def fill_z(j, carry):
        zb_v[j, :] = z16
        return carry
    lax.fori_loop(0, ZR, fill_z, 0)

    for q in range(8):
        pltpu.sync_copy(zb_v, degsp.at[pl.ds(sid * RPT + q * ZR, ZR)])
    pltpu.sync_copy(pk_hbm.at[w], pk_v)
    plsc.subcore_barrier()

    def chunk(j, carry):
        for k in range(CH // 16):
            sl = pl.ds(16 * k, 16)
            idx_v[sl] = lax.shift_right_logical(pk_v[j, sl], SB)
        pltpu.sync_copy(ones_v, degsp.at[idx_v], add=True)
        return carry
    lax.fori_loop(0, CPT, chunk, 0)

    plsc.subcore_barrier()
    pltpu.sync_copy(degsp.at[pl.ds(sid * RPT, RPT)],
                    degw_hbm.at[cid, pl.ds(sid * RPT, RPT)])


@functools.partial(
    pl.kernel,
    out_type=jax.ShapeDtypeStruct((NC, NSP, D), jnp.float32),
    mesh=_mesh,
    scratch_types=[
        pltpu.VMEM((CPT, CH), jnp.int32),
        pltpu.VMEM((CH,), jnp.int32),
        pltpu.VMEM((CH,), jnp.int32),
        pltpu.VMEM((CH, D), jnp.float32),
        pltpu.VMEM((ZR, D), jnp.float32),
        pltpu.SemaphoreType.DMA,
        pltpu.VMEM_SHARED((NSP, D), jnp.float32),
    ],
)
def _sc_aggregate(xp_hbm, pk_hbm, parts_hbm,
                  pk_v, src_v, dst_v, buf, zb_v, sem, acc):
    cid = lax.axis_index("c")
    sid = lax.axis_index("s")
    w = cid * NS + sid
    z16 = jnp.zeros((16,), jnp.float32)

    def fill_z(j, carry):
        for k in range(D // 16):
            zb_v[j, pl.ds(16 * k, 16)] = z16
        return carry
    lax.fori_loop(0, ZR, fill_z, 0)

    for q in range(8):
        pltpu.sync_copy(zb_v, acc.at[pl.ds(sid * RPT + q * ZR, ZR)])
    pltpu.sync_copy(pk_hbm.at[w], pk_v)
    plsc.subcore_barrier()

    def chunk(j, carry):
        for k in range(CH // 16):
            sl = pl.ds(16 * k, 16)
            v = pk_v[j, sl]
            src_v[sl] = lax.bitwise_and(v, SM)
            dst_v[sl] = lax.shift_right_logical(v, SB)
        pltpu.async_copy(xp_hbm.at[src_v], buf, sem).wait()
        pltpu.sync_copy(buf, acc.at[dst_v], add=True)
        return carry
    lax.fori_loop(0, CPT, chunk, 0)

    plsc.subcore_barrier()
    pltpu.sync_copy(acc.at[pl.ds(sid * RPT, RPT)],
                    parts_hbm.at[cid, pl.ds(sid * RPT, RPT)])


_R1 = 2528


def _t1_body(degw_ref, x_ref, xp_ref, dinv_ref):
    deg = jnp.sum(degw_ref[...], axis=(0, 2))
    dinv = lax.rsqrt(deg)[:, None]
    dinv_ref[...] = dinv
    xp_ref[...] = x_ref[...] * dinv


_tc_scale = pl.pallas_call(
    _t1_body,
    grid=(NSP // _R1,),
    in_specs=[
        pl.BlockSpec((NC, _R1, 16), lambda i: (0, i, 0)),
        pl.BlockSpec((_R1, D), lambda i: (i, 0)),
    ],
    out_specs=[
        pl.BlockSpec((_R1, D), lambda i: (i, 0)),
        pl.BlockSpec((_R1, 1), lambda i: (i, 0)),
    ],
    out_shape=[
        jax.ShapeDtypeStruct((NSP, D), jnp.float32),
        jax.ShapeDtypeStruct((NSP, 1), jnp.float32),
    ],
)

_R2 = 2000


def _t2_body(parts_ref, dinv_ref, out_ref):
    dinv = dinv_ref[...]
    out_ref[...] = jnp.maximum(
        (parts_ref[0] + parts_ref[1]) * dinv, jnp.float32(0))


_tc_combine = pl.pallas_call(
    _t2_body,
    grid=(N // _R2,),
    in_specs=[
        pl.BlockSpec((NC, _R2, D), lambda i: (0, i, 0)),
        pl.BlockSpec((_R2, 1), lambda i: (i, 0)),
    ],
    out_specs=pl.BlockSpec((_R2, D), lambda i: (i, 0)),
    out_shape=jax.ShapeDtypeStruct((N, D), jnp.float32),
)


@jax.jit
def kernel(x, edge_index):
    src = edge_index[0].astype(jnp.int32)
    dst = edge_index[1].astype(jnp.int32)
    pad_e = EPAD - E
    pk = jnp.left_shift(dst, SB) | src
    pk_p = jnp.concatenate(
        [pk, jnp.full((pad_e,), N << SB, jnp.int32)]).reshape(NT, CPT, CH)
    x_p = jnp.pad(x, ((0, NSP - N), (0, 0)))

    degw = _sc_degree(pk_p)
    xp, dinv = _tc_scale(degw, x_p)
    parts = _sc_aggregate(xp, pk_p)
    return _tc_combine(parts, dinv)


# speedup vs baseline: 11.1773x; 1.1002x over previous
import functools

import jax
import jax.numpy as jnp
from jax import lax
from jax.experimental import pallas as pl
from jax.experimental.pallas import tpu as pltpu
from jax.experimental.pallas import tpu_sc as plsc

N = 10000
D = 128
E = 320000
NC, NS = 2, 16
NT = NC * NS
CH = 128
CPT = 80
EPAD = NT * CPT * CH
NSP = 10112
RPT = NSP // NS
ZR = RPT // 8
SB = 14
SM = (1 << SB) - 1

_mesh = plsc.VectorSubcoreMesh(
    core_axis_name="c", subcore_axis_name="s", num_cores=NC, num_subcores=NS)


@functools.partial(
    pl.kernel,
    out_type=jax.ShapeDtypeStruct((NC, NSP, 16), jnp.float32),
    mesh=_mesh,
    scratch_types=[
        pltpu.VMEM((CPT, CH), jnp.int32),
        pltpu.VMEM((CH,), jnp.int32),
        pltpu.VMEM((CH, 16), jnp.float32),
        pltpu.VMEM((ZR, 16), jnp.float32),
        pltpu.VMEM_SHARED((NSP, 16), jnp.float32),
    ],
)
def _sc_degree(pk_hbm, degw_hbm, pk_v, idx_v, ones_v, zb_v, degsp):
    cid = lax.axis_index("c")
    sid = lax.axis_index("s")
    w = cid * NS + sid
    e0 = jnp.where(lax.iota(jnp.int32, 16) == 0,
                   jnp.float32(1), jnp.float32(0))
    z16 = jnp.zeros((16,), jnp.float32)

    def fill_ones(j, carry):
        ones_v[j, :] = e0
        return carry
    lax.fori_loop(0, CH, fill_ones, 0)

    def fill_z(j, carry):
        zb_v[j, :] = z16
        return carry
    lax.fori_loop(0, ZR, fill_z, 0)

    for q in range(8):
        pltpu.sync_copy(zb_v, degsp.at[pl.ds(sid * RPT + q * ZR, ZR)])
    pltpu.sync_copy(pk_hbm.at[w], pk_v)
    plsc.subcore_barrier()

    def chunk(j, carry):
        for k in range(CH // 16):
            sl = pl.ds(16 * k, 16)
            idx_v[sl] = lax.shift_right_logical(pk_v[j, sl], SB)
        pltpu.sync_copy(ones_v, degsp.at[idx_v], add=True)
        return carry
    lax.fori_loop(0, CPT, chunk, 0)

    plsc.subcore_barrier()
    pltpu.sync_copy(degsp.at[pl.ds(sid * RPT, RPT)],
                    degw_hbm.at[cid, pl.ds(sid * RPT, RPT)])


@functools.partial(
    pl.kernel,
    out_type=jax.ShapeDtypeStruct((NC, NSP, D), jnp.float32),
    mesh=_mesh,
    scratch_types=[
        pltpu.VMEM((CPT, CH), jnp.int32),
        pltpu.VMEM((CH,), jnp.int32),
        pltpu.VMEM((CH,), jnp.int32),
        pltpu.VMEM((CH, D), jnp.float32),
        pltpu.VMEM((CH,), jnp.int32),
        pltpu.VMEM((CH,), jnp.int32),
        pltpu.VMEM((CH, D), jnp.float32),
        pltpu.SemaphoreType.DMA,
        pltpu.SemaphoreType.DMA,
        pltpu.VMEM_SHARED((NSP, D), jnp.float32),
    ],
)
def _sc_aggregate(xp_hbm, pk_hbm, parts_hbm,
                  pk_v, src0, dst0, buf0, src1, dst1, buf1,
                  gsem0, gsem1, acc):
    cid = lax.axis_index("c")
    sid = lax.axis_index("s")
    w = cid * NS + sid
    z16 = jnp.zeros((16,), jnp.float32)

    def fill_z(j, carry):
        for k in range(D // 16):
            buf0[j, pl.ds(16 * k, 16)] = z16
        return carry
    lax.fori_loop(0, CH, fill_z, 0)

    for q in range(4):
        pltpu.sync_copy(buf0, acc.at[pl.ds(sid * RPT + q * CH, CH)])
    pltpu.sync_copy(buf0.at[pl.ds(0, RPT - 4 * CH)],
                    acc.at[pl.ds(sid * RPT + 4 * CH, RPT - 4 * CH)])
    pltpu.sync_copy(pk_hbm.at[w], pk_v)
    plsc.subcore_barrier()

    def unpack(j, sv, dv):
        for k in range(CH // 16):
            sl = pl.ds(16 * k, 16)
            v = pk_v[j, sl]
            sv[sl] = lax.bitwise_and(v, SM)
            dv[sl] = lax.shift_right_logical(v, SB)

    unpack(0, src0, dst0)
    pltpu.async_copy(xp_hbm.at[src0], buf0, gsem0)

    def step(j, carry):
        unpack(2 * j + 1, src1, dst1)
        pltpu.async_copy(xp_hbm.at[src1], buf1, gsem1)
        pltpu.make_async_copy(xp_hbm.at[src0], buf0, gsem0).wait()
        pltpu.sync_copy(buf0, acc.at[dst0], add=True)

        @pl.when(j + 1 < CPT // 2)
        def _():
            unpack(2 * j + 2, src0, dst0)
            pltpu.async_copy(xp_hbm.at[src0], buf0, gsem0)

        pltpu.make_async_copy(xp_hbm.at[src1], buf1, gsem1).wait()
        pltpu.sync_copy(buf1, acc.at[dst1], add=True)
        return carry
    lax.fori_loop(0, CPT // 2, step, 0)

    plsc.subcore_barrier()
    pltpu.sync_copy(acc.at[pl.ds(sid * RPT, RPT)],
                    parts_hbm.at[cid, pl.ds(sid * RPT, RPT)])


_R1 = 2528


def _t1_body(degw_ref, x_ref, xp_ref, dinv_ref):
    deg = jnp.sum(degw_ref[...], axis=(0, 2))
    dinv = lax.rsqrt(deg)[:, None]
    dinv_ref[...] = dinv
    xp_ref[...] = x_ref[...] * dinv


_tc_scale = pl.pallas_call(
    _t1_body,
    grid=(NSP // _R1,),
    in_specs=[
        pl.BlockSpec((NC, _R1, 16), lambda i: (0, i, 0)),
        pl.BlockSpec((_R1, D), lambda i: (i, 0)),
    ],
    out_specs=[
        pl.BlockSpec((_R1, D), lambda i: (i, 0)),
        pl.BlockSpec((_R1, 1), lambda i: (i, 0)),
    ],
    out_shape=[
        jax.ShapeDtypeStruct((NSP, D), jnp.float32),
        jax.ShapeDtypeStruct((NSP, 1), jnp.float32),
    ],
)

_R2 = 2000


def _t2_body(parts_ref, dinv_ref, out_ref):
    dinv = dinv_ref[...]
    out_ref[...] = jnp.maximum(
        (parts_ref[0] + parts_ref[1]) * dinv, jnp.float32(0))


_tc_combine = pl.pallas_call(
    _t2_body,
    grid=(N // _R2,),
    in_specs=[
        pl.BlockSpec((NC, _R2, D), lambda i: (0, i, 0)),
        pl.BlockSpec((_R2, 1), lambda i: (i, 0)),
    ],
    out_specs=pl.BlockSpec((_R2, D), lambda i: (i, 0)),
    out_shape=jax.ShapeDtypeStruct((N, D), jnp.float32),
)


@jax.jit
def kernel(x, edge_index):
    src = edge_index[0].astype(jnp.int32)
    dst = edge_index[1].astype(jnp.int32)
    pad_e = EPAD - E
    pk = jnp.left_shift(dst, SB) | src
    pk_p = jnp.concatenate(
        [pk, jnp.full((pad_e,), N << SB, jnp.int32)]).reshape(NT, CPT, CH)
    x_p = jnp.pad(x, ((0, NSP - N), (0, 0)))

    degw = _sc_degree(pk_p)
    xp, dinv = _tc_scale(degw, x_p)
    parts = _sc_aggregate(xp, pk_p)
    return _tc_combine(parts, dinv)
